# Initial kernel scaffold; baseline (speedup 1.0000x reference)
import jax
import jax.numpy as jnp
from jax.experimental import pallas as pl


def kernel(rel_idx, position_idx, rel_table, pos_table):
    raise NotImplementedError("write your pallas kernel here")


# speedup vs baseline: 8.2990x; 8.2990x over previous
import functools

import jax
import jax.numpy as jnp
from jax import lax
from jax.experimental import pallas as pl
from jax.experimental.pallas import tpu as pltpu
from jax.experimental.pallas import tpu_sc as plsc

B = 16384
L = 200
REL_DIM = 32
RP_DIM = 32
OUT_DIM = REL_DIM + RP_DIM

N = B * L
NUM_WORKERS = 32
PER_W = N // NUM_WORKERS
CHUNK = 512
STEPS = PER_W // CHUNK


def _gather_body(rel_idx_hbm, pos_idx_hbm, rel_tab_hbm, pos_tab_hbm,
                 out_hbm,
                 idx_rel_v, idx_pos_v, rel_rows_v, pos_rows_v,
                 idx_sem, gat_sem, out_sem):
    c = lax.axis_index("c")
    s = lax.axis_index("s")
    wid = s * 2 + c
    base_w = wid * PER_W

    def step(i, _):
        base = base_w + i * CHUNK
        cp_r = pltpu.make_async_copy(
            rel_idx_hbm.at[pl.ds(base, CHUNK)], idx_rel_v, idx_sem)
        cp_p = pltpu.make_async_copy(
            pos_idx_hbm.at[pl.ds(base, CHUNK)], idx_pos_v, idx_sem)
        cp_r.start()
        cp_p.start()
        cp_r.wait()
        cp_p.wait()
        g_r = pltpu.make_async_copy(
            rel_tab_hbm.at[idx_rel_v], rel_rows_v, gat_sem)
        g_p = pltpu.make_async_copy(
            pos_tab_hbm.at[idx_pos_v], pos_rows_v, gat_sem)
        g_r.start()
        g_p.start()
        g_r.wait()
        g_p.wait()
        o_r = pltpu.make_async_copy(
            rel_rows_v, out_hbm.at[pl.ds(base, CHUNK), pl.ds(0, REL_DIM)],
            out_sem)
        o_p = pltpu.make_async_copy(
            pos_rows_v, out_hbm.at[pl.ds(base, CHUNK), pl.ds(REL_DIM, RP_DIM)],
            out_sem)
        o_r.start()
        o_p.start()
        o_r.wait()
        o_p.wait()
        return ()

    lax.fori_loop(0, STEPS, step, (), unroll=False)


@jax.jit
def _tree_embedding(rel_idx_flat, pos_idx_flat, rel_table, pos_table):
    mesh = plsc.VectorSubcoreMesh(core_axis_name="c", subcore_axis_name="s")
    kern = pl.kernel(
        _gather_body,
        out_type=jax.ShapeDtypeStruct((N, OUT_DIM), jnp.float32),
        mesh=mesh,
        compiler_params=pltpu.CompilerParams(use_tc_tiling_on_sc=False),
        scratch_types=[
            pltpu.VMEM((CHUNK,), jnp.int32),
            pltpu.VMEM((CHUNK,), jnp.int32),
            pltpu.VMEM((CHUNK, REL_DIM), jnp.float32),
            pltpu.VMEM((CHUNK, RP_DIM), jnp.float32),
            pltpu.SemaphoreType.DMA,
            pltpu.SemaphoreType.DMA,
            pltpu.SemaphoreType.DMA,
        ],
    )
    return kern(rel_idx_flat, pos_idx_flat, rel_table, pos_table)


def kernel(rel_idx, position_idx, rel_table, pos_table):
    out = _tree_embedding(rel_idx.reshape(N), position_idx.reshape(N),
                          rel_table, pos_table)
    return out.reshape(B, L, OUT_DIM)


# speedup vs baseline: 8.3179x; 1.0023x over previous
import jax
import jax.numpy as jnp
from jax import lax
from jax.experimental import pallas as pl
from jax.experimental.pallas import tpu as pltpu
from jax.experimental.pallas import tpu_sc as plsc

B = 16384
L = 200
REL_DIM = 32
RP_DIM = 32
OUT_DIM = REL_DIM + RP_DIM

N = B * L
NUM_WORKERS = 32
PER_W = N // NUM_WORKERS
CHUNK = 800
STEPS = PER_W // CHUNK
NBUF = 2


def _gather_body(rel_idx_hbm, pos_idx_hbm, rel_tab_hbm, pos_tab_hbm,
                 out_hbm,
                 idx_rel0, idx_rel1, idx_pos0, idx_pos1,
                 rel_rows0, rel_rows1, pos_rows0, pos_rows1,
                 idx_sem0, idx_sem1, gat_sem0, gat_sem1,
                 out_sem0, out_sem1):
    c = lax.axis_index("c")
    s = lax.axis_index("s")
    wid = s * 2 + c
    base_w = wid * PER_W

    idx_rel = (idx_rel0, idx_rel1)
    idx_pos = (idx_pos0, idx_pos1)
    rel_rows = (rel_rows0, rel_rows1)
    pos_rows = (pos_rows0, pos_rows1)
    idx_sem = (idx_sem0, idx_sem1)
    gat_sem = (gat_sem0, gat_sem1)
    out_sem = (out_sem0, out_sem1)

    def idx_copies(i, b):
        ii = lax.min(i, STEPS - 1)
        base = base_w + ii * CHUNK
        return (pltpu.make_async_copy(
                    rel_idx_hbm.at[pl.ds(base, CHUNK)], idx_rel[b],
                    idx_sem[b]),
                pltpu.make_async_copy(
                    pos_idx_hbm.at[pl.ds(base, CHUNK)], idx_pos[b],
                    idx_sem[b]))

    def gathers(b):
        return (pltpu.make_async_copy(
                    rel_tab_hbm.at[idx_rel[b]], rel_rows[b], gat_sem[b]),
                pltpu.make_async_copy(
                    pos_tab_hbm.at[idx_pos[b]], pos_rows[b], gat_sem[b]))

    def writes(i, b):
        base = base_w + i * CHUNK
        return (pltpu.make_async_copy(
                    rel_rows[b],
                    out_hbm.at[pl.ds(base, CHUNK), pl.ds(0, REL_DIM)],
                    out_sem[b]),
                pltpu.make_async_copy(
                    pos_rows[b],
                    out_hbm.at[pl.ds(base, CHUNK), pl.ds(REL_DIM, RP_DIM)],
                    out_sem[b]))

    def block(i0, first):
        for b in range(NBUF):
            i = i0 + b
            for cp in idx_copies(i, b):
                cp.wait()
            if not first:
                for cp in writes(i - NBUF, b):
                    cp.wait()
            for cp in gathers(b):
                cp.start()
        for b in range(NBUF):
            i = i0 + b
            for cp in gathers(b):
                cp.wait()
            for cp in writes(i, b):
                cp.start()
            for cp in idx_copies(i + NBUF, b):
                cp.start()

    for b in range(NBUF):
        for cp in idx_copies(b, b):
            cp.start()
    block(0, first=True)
    lax.fori_loop(1, STEPS // NBUF,
                  lambda og, _: (block(og * NBUF, first=False), ())[1],
                  (), unroll=False)
    for b in range(NBUF):
        for cp in writes(STEPS - NBUF + b, b):
            cp.wait()
        for cp in idx_copies(0, b):
            cp.wait()


@jax.jit
def _tree_embedding(rel_idx_flat, pos_idx_flat, rel_table, pos_table):
    mesh = plsc.VectorSubcoreMesh(core_axis_name="c", subcore_axis_name="s")
    kern = pl.kernel(
        _gather_body,
        out_type=jax.ShapeDtypeStruct((N, OUT_DIM), jnp.float32),
        mesh=mesh,
        compiler_params=pltpu.CompilerParams(use_tc_tiling_on_sc=False),
        scratch_types=[
            pltpu.VMEM((CHUNK,), jnp.int32),
            pltpu.VMEM((CHUNK,), jnp.int32),
            pltpu.VMEM((CHUNK,), jnp.int32),
            pltpu.VMEM((CHUNK,), jnp.int32),
            pltpu.VMEM((CHUNK, REL_DIM), jnp.float32),
            pltpu.VMEM((CHUNK, REL_DIM), jnp.float32),
            pltpu.VMEM((CHUNK, RP_DIM), jnp.float32),
            pltpu.VMEM((CHUNK, RP_DIM), jnp.float32),
            pltpu.SemaphoreType.DMA,
            pltpu.SemaphoreType.DMA,
            pltpu.SemaphoreType.DMA,
            pltpu.SemaphoreType.DMA,
            pltpu.SemaphoreType.DMA,
            pltpu.SemaphoreType.DMA,
        ],
    )
    return kern(rel_idx_flat, pos_idx_flat, rel_table, pos_table)


def kernel(rel_idx, position_idx, rel_table, pos_table):
    out = _tree_embedding(rel_idx.reshape(N), position_idx.reshape(N),
                          rel_table, pos_table)
    return out.reshape(B, L, OUT_DIM)
